# Initial kernel scaffold; baseline (speedup 1.0000x reference)
import jax
import jax.numpy as jnp
from jax.experimental import pallas as pl


def kernel(features, adj, nodes, W1, W2):
    raise NotImplementedError("write your pallas kernel here")


# speedup vs baseline: 3.9494x; 3.9494x over previous
import functools

import jax
import jax.numpy as jnp
import numpy as np
from jax import lax
from jax.experimental import pallas as pl
from jax.experimental.pallas import tpu as pltpu
from jax.experimental.pallas import tpu_sc as plsc

_NC = 2
_NS = 16
_NW = _NC * _NS
_L = 16


def _sc_agg1(features, adjx, nodes32, *, B, H, D, N):
    G = B * H
    BPW = B // _NW
    GPW = BPW * H
    FPW = GPW * H
    CH = 16
    NCHUNK = GPW // CH
    R = CH * H

    mesh = plsc.VectorSubcoreMesh(core_axis_name="c", subcore_axis_name="s")

    @functools.partial(
        pl.kernel,
        out_type=jax.ShapeDtypeStruct((G, D), jnp.float32),
        mesh=mesh,
        compiler_params=pltpu.CompilerParams(
            needs_layout_passes=False, use_tc_tiling_on_sc=False),
        scratch_types=[
            pltpu.VMEM((BPW,), jnp.int32),
            pltpu.VMEM((BPW, 16), jnp.int32),
            pltpu.VMEM((BPW * 16,), jnp.int32),
            pltpu.VMEM((BPW * 16, 16), jnp.int32),
            pltpu.VMEM((FPW,), jnp.int32),
            pltpu.VMEM((R, D), jnp.float32),
            pltpu.VMEM((CH, D), jnp.float32),
            pltpu.SemaphoreType.DMA,
            pltpu.SemaphoreType.DMA,
        ],
    )
    def k(features_hbm, adjx_hbm, nodes_hbm, out_hbm,
          nodes_v, nb2x_v, hood2p_v, nb1x_v, fidx_v,
          rows_v, outbuf_v, sem, sem2):
        wid = lax.axis_index("s") * _NC + lax.axis_index("c")
        nbase = wid * BPW
        gbase = wid * GPW

        pltpu.sync_copy(nodes_hbm.at[pl.ds(nbase, BPW)], nodes_v)

        pltpu.async_copy(adjx_hbm.at[nodes_v], nb2x_v, sem).wait()

        def h2_body(kk, _):
            hood2p_v[pl.ds(kk * 16, 16)] = nb2x_v[kk, :]
            return 0

        lax.fori_loop(0, BPW, h2_body, 0)

        pltpu.async_copy(adjx_hbm.at[hood2p_v], nb1x_v, sem).wait()

        iot = lax.iota(jnp.int32, 16)
        msk = iot < H

        def fx_body(kk, _):
            for i in range(H):
                v = nb1x_v[kk * 16 + i, :]
                pos = kk * (H * H) + i * H + iot
                plsc.store_scatter(fidx_v, [pos], v, mask=msk)
            return 0

        lax.fori_loop(0, BPW, fx_body, 0)

        def chunk_body(c, _):
            pltpu.async_copy(
                features_hbm.at[fidx_v.at[pl.ds(c * R, R)]], rows_v, sem2
            ).wait()

            def g_body(g, _):
                base = g * H
                for kk in range(D // _L):
                    a = rows_v[base, pl.ds(kk * _L, _L)]
                    for j in range(1, H):
                        a = a + rows_v[base + j, pl.ds(kk * _L, _L)]
                    outbuf_v[g, pl.ds(kk * _L, _L)] = a * (1.0 / H)
                return 0

            lax.fori_loop(0, CH, g_body, 0)
            pltpu.sync_copy(outbuf_v, out_hbm.at[pl.ds(gbase + c * CH, CH)])
            return 0

        lax.fori_loop(0, NCHUNK, chunk_body, 0)

    return k(features, adjx, nodes32)


def _tc_encode(agg1, W1t, W2t, *, B, H, D, E):
    BBLK = 256
    RBLK = BBLK * H

    def body(x_ref, w1t_ref, w2t_ref, out_ref):
        x = x_ref[...]
        h1 = jnp.maximum(
            jnp.dot(x, w1t_ref[...], preferred_element_type=jnp.float32), 0.0)
        a2 = jnp.mean(h1.reshape(BBLK, H, E), axis=1)
        h2 = jnp.maximum(
            jnp.dot(a2, w2t_ref[...], preferred_element_type=jnp.float32), 0.0)
        out_ref[...] = h2.T

    return pl.pallas_call(
        body,
        grid=(B // BBLK,),
        in_specs=[
            pl.BlockSpec((RBLK, D), lambda i: (i, 0)),
            pl.BlockSpec((D, E), lambda i: (0, 0)),
            pl.BlockSpec((E, E), lambda i: (0, 0)),
        ],
        out_specs=pl.BlockSpec((E, BBLK), lambda i: (0, i)),
        out_shape=jax.ShapeDtypeStruct((E, B), jnp.float32),
    )(agg1, W1t, W2t)


def kernel(features, adj, nodes, W1, W2):
    N, S = adj.shape
    B = nodes.shape[0]
    D = features.shape[1]
    E = W1.shape[0]
    H = S + 1

    adj32 = adj.astype(jnp.int32)
    selfcol = jnp.arange(N, dtype=jnp.int32)[:, None]
    adjx = jnp.concatenate(
        [adj32, selfcol, jnp.zeros((N, 16 - S - 1), jnp.int32)], axis=1)
    nodes32 = nodes.astype(jnp.int32)

    agg1 = _sc_agg1(features, adjx, nodes32, B=B, H=H, D=D, N=N)
    return _tc_encode(agg1, W1.T, W2.T, B=B, H=H, D=D, E=E)


# speedup vs baseline: 5.2023x; 1.3172x over previous
import functools

import jax
import jax.numpy as jnp
import numpy as np
from jax import lax
from jax.experimental import pallas as pl
from jax.experimental.pallas import tpu as pltpu
from jax.experimental.pallas import tpu_sc as plsc

_NC = 2
_NS = 16
_NW = _NC * _NS
_L = 16


def _sc_agg1(features, adjx, nodes32, *, B, H, D, N):
    G = B * H
    BPW = B // _NW
    GPW = BPW * H
    FPW = GPW * H
    CH = 16
    NCHUNK = GPW // CH
    R = CH * H

    mesh = plsc.VectorSubcoreMesh(core_axis_name="c", subcore_axis_name="s")

    @functools.partial(
        pl.kernel,
        out_type=jax.ShapeDtypeStruct((G, D), jnp.float32),
        mesh=mesh,
        compiler_params=pltpu.CompilerParams(
            needs_layout_passes=False, use_tc_tiling_on_sc=False),
        scratch_types=[
            pltpu.VMEM((BPW,), jnp.int32),
            pltpu.VMEM((BPW, 16), jnp.int32),
            pltpu.VMEM((BPW * 16,), jnp.int32),
            pltpu.VMEM((BPW * 16, 16), jnp.int32),
            pltpu.VMEM((FPW,), jnp.int32),
            pltpu.VMEM((R, D), jnp.float32),
            pltpu.VMEM((R, D), jnp.float32),
            pltpu.VMEM((CH, D), jnp.float32),
            pltpu.VMEM((CH, D), jnp.float32),
            pltpu.SemaphoreType.DMA,
            pltpu.SemaphoreType.DMA,
            pltpu.SemaphoreType.DMA,
            pltpu.SemaphoreType.DMA,
            pltpu.SemaphoreType.DMA,
        ],
    )
    def k(features_hbm, adjx_hbm, nodes_hbm, out_hbm,
          nodes_v, nb2x_v, hood2p_v, nb1x_v, fidx_v,
          rows_v0, rows_v1, outbuf_v0, outbuf_v1,
          sem, gsem0, gsem1, osem0, osem1):
        wid = lax.axis_index("s") * _NC + lax.axis_index("c")
        nbase = wid * BPW
        gbase = wid * GPW

        pltpu.sync_copy(nodes_hbm.at[pl.ds(nbase, BPW)], nodes_v)

        pltpu.async_copy(adjx_hbm.at[nodes_v], nb2x_v, sem).wait()

        def h2_body(kk, _):
            hood2p_v[pl.ds(kk * 16, 16)] = nb2x_v[kk, :]
            return 0

        lax.fori_loop(0, BPW, h2_body, 0)

        pltpu.async_copy(adjx_hbm.at[hood2p_v], nb1x_v, sem).wait()

        iot = lax.iota(jnp.int32, 16)
        msk = iot < H

        def fx_body(kk, _):
            for i in range(H):
                v = nb1x_v[kk * 16 + i, :]
                pos = kk * (H * H) + i * H + iot
                plsc.store_scatter(fidx_v, [pos], v, mask=msk)
            return 0

        lax.fori_loop(0, BPW, fx_body, 0)

        bufs = ((rows_v0, gsem0, outbuf_v0, osem0),
                (rows_v1, gsem1, outbuf_v1, osem1))

        def gather_desc(c, buf, gsem):
            return pltpu.make_async_copy(
                features_hbm.at[fidx_v.at[pl.ds(c * R, R)]], buf, gsem)

        def out_desc(c, obuf, osem):
            return pltpu.make_async_copy(
                obuf, out_hbm.at[pl.ds(gbase + c * CH, CH)], osem)

        gather_desc(0, rows_v0, gsem0).start()

        def chunk_pair(cc, _):
            for b in range(2):
                c = cc * 2 + b
                buf, gsem, obuf, osem = bufs[b]
                nbuf, ngsem = bufs[1 - b][0], bufs[1 - b][1]

                @pl.when(c + 1 < NCHUNK)
                def _():
                    gather_desc(c + 1, nbuf, ngsem).start()

                gather_desc(c, buf, gsem).wait()

                @pl.when(c >= 2)
                def _():
                    out_desc(c - 2, obuf, osem).wait()

                def g_body(g, _):
                    base = g * H
                    for kk in range(D // _L):
                        a = buf[base, pl.ds(kk * _L, _L)]
                        for j in range(1, H):
                            a = a + buf[base + j, pl.ds(kk * _L, _L)]
                        obuf[g, pl.ds(kk * _L, _L)] = a * (1.0 / H)
                    return 0

                lax.fori_loop(0, CH, g_body, 0)
                out_desc(c, obuf, osem).start()
            return 0

        lax.fori_loop(0, NCHUNK // 2, chunk_pair, 0)
        out_desc(NCHUNK - 2, outbuf_v0, osem0).wait()
        out_desc(NCHUNK - 1, outbuf_v1, osem1).wait()

    return k(features, adjx, nodes32)


def _tc_encode(agg1, W1t, W2t, *, B, H, D, E):
    BBLK = 256
    RBLK = BBLK * H

    def body(x_ref, w1t_ref, w2t_ref, out_ref):
        x = x_ref[...]
        h1 = jnp.maximum(
            jnp.dot(x, w1t_ref[...], preferred_element_type=jnp.float32), 0.0)
        a2 = jnp.mean(h1.reshape(BBLK, H, E), axis=1)
        h2 = jnp.maximum(
            jnp.dot(a2, w2t_ref[...], preferred_element_type=jnp.float32), 0.0)
        out_ref[...] = h2.T

    return pl.pallas_call(
        body,
        grid=(B // BBLK,),
        in_specs=[
            pl.BlockSpec((RBLK, D), lambda i: (i, 0)),
            pl.BlockSpec((D, E), lambda i: (0, 0)),
            pl.BlockSpec((E, E), lambda i: (0, 0)),
        ],
        out_specs=pl.BlockSpec((E, BBLK), lambda i: (0, i)),
        out_shape=jax.ShapeDtypeStruct((E, B), jnp.float32),
    )(agg1, W1t, W2t)


def kernel(features, adj, nodes, W1, W2):
    N, S = adj.shape
    B = nodes.shape[0]
    D = features.shape[1]
    E = W1.shape[0]
    H = S + 1

    adj32 = adj.astype(jnp.int32)
    selfcol = jnp.arange(N, dtype=jnp.int32)[:, None]
    adjx = jnp.concatenate(
        [adj32, selfcol, jnp.zeros((N, 16 - S - 1), jnp.int32)], axis=1)
    nodes32 = nodes.astype(jnp.int32)

    agg1 = _sc_agg1(features, adjx, nodes32, B=B, H=H, D=D, N=N)
    return _tc_encode(agg1, W1.T, W2.T, B=B, H=H, D=D, E=E)


# speedup vs baseline: 6.3629x; 1.2231x over previous
import functools

import jax
import jax.numpy as jnp
import numpy as np
from jax import lax
from jax.experimental import pallas as pl
from jax.experimental.pallas import tpu as pltpu
from jax.experimental.pallas import tpu_sc as plsc

_NC = 2
_NS = 16
_NW = _NC * _NS
_L = 16


def _sc_agg1(features, adjx, nodes32, *, B, H, D, N):
    G = B * H
    BPW = B // _NW
    GPW = BPW * H
    FPW = GPW * H

    mesh = plsc.VectorSubcoreMesh(core_axis_name="c", subcore_axis_name="s")

    @functools.partial(
        pl.kernel,
        out_type=jax.ShapeDtypeStruct((G, D), jnp.float32),
        mesh=mesh,
        compiler_params=pltpu.CompilerParams(
            needs_layout_passes=False, use_tc_tiling_on_sc=False),
        scratch_types=[
            pltpu.VMEM((BPW,), jnp.int32),
            pltpu.VMEM((BPW, 16), jnp.int32),
            pltpu.VMEM((BPW * 16,), jnp.int32),
            pltpu.VMEM((BPW * 16, 16), jnp.int32),
            pltpu.VMEM((FPW,), jnp.int32),
            pltpu.VMEM((GPW, D), jnp.float32),
            pltpu.SemaphoreType.DMA,
            pltpu.SemaphoreType.DMA,
        ],
    )
    def k(features_hbm, adjx_hbm, nodes_hbm, out_hbm,
          nodes_v, nb2x_v, hood2p_v, nb1x_v, fidx_v, acc_v, sem, sem2):
        wid = lax.axis_index("s") * _NC + lax.axis_index("c")
        nbase = wid * BPW
        gbase = wid * GPW

        pltpu.sync_copy(nodes_hbm.at[pl.ds(nbase, BPW)], nodes_v)

        pltpu.async_copy(adjx_hbm.at[nodes_v], nb2x_v, sem).wait()

        def h2_body(kk, _):
            hood2p_v[pl.ds(kk * 16, 16)] = nb2x_v[kk, :]
            return 0

        lax.fori_loop(0, BPW, h2_body, 0)

        pltpu.async_copy(adjx_hbm.at[hood2p_v], nb1x_v, sem).wait()

        iot = lax.iota(jnp.int32, 16)
        msk = iot < H

        def fx_body(kk, _):
            for i in range(H):
                v = nb1x_v[kk * 16 + i, :]
                pos = iot * GPW + (kk * H + i)
                plsc.store_scatter(fidx_v, [pos], v, mask=msk)
            return 0

        lax.fori_loop(0, BPW, fx_body, 0)

        pltpu.async_copy(
            features_hbm.at[fidx_v.at[pl.ds(0, GPW)]], acc_v, sem2).wait()
        for j in range(1, H):
            pltpu.async_copy(
                features_hbm.at[fidx_v.at[pl.ds(j * GPW, GPW)]], acc_v, sem2,
                add=True).wait()

        pltpu.sync_copy(acc_v, out_hbm.at[pl.ds(gbase, GPW)])

    return k(features, adjx, nodes32)


def _tc_encode(agg1, W1t, W2t, *, B, H, D, E):
    BBLK = 256
    RBLK = BBLK * H

    def body(x_ref, w1t_ref, w2t_ref, out_ref):
        x = x_ref[...]
        h1 = jnp.maximum(
            jnp.dot(x, w1t_ref[...], preferred_element_type=jnp.float32), 0.0)
        a2 = jnp.sum(h1.reshape(BBLK, H, E), axis=1)
        h2 = jnp.maximum(
            jnp.dot(a2, w2t_ref[...], preferred_element_type=jnp.float32), 0.0)
        out_ref[...] = h2.T

    return pl.pallas_call(
        body,
        grid=(B // BBLK,),
        in_specs=[
            pl.BlockSpec((RBLK, D), lambda i: (i, 0)),
            pl.BlockSpec((D, E), lambda i: (0, 0)),
            pl.BlockSpec((E, E), lambda i: (0, 0)),
        ],
        out_specs=pl.BlockSpec((E, BBLK), lambda i: (0, i)),
        out_shape=jax.ShapeDtypeStruct((E, B), jnp.float32),
    )(agg1, W1t, W2t)


def kernel(features, adj, nodes, W1, W2):
    N, S = adj.shape
    B = nodes.shape[0]
    D = features.shape[1]
    E = W1.shape[0]
    H = S + 1

    adj32 = adj.astype(jnp.int32)
    selfcol = jnp.arange(N, dtype=jnp.int32)[:, None]
    adjx = jnp.concatenate(
        [adj32, selfcol, jnp.zeros((N, 16 - S - 1), jnp.int32)], axis=1)
    nodes32 = nodes.astype(jnp.int32)

    agg1 = _sc_agg1(features, adjx, nodes32, B=B, H=H, D=D, N=N)
    return _tc_encode(agg1, W1.T / H, W2.T / H, B=B, H=H, D=D, E=E)
